# baseline (speedup 1.0000x reference)
import functools

import jax
import jax.numpy as jnp
from jax import lax
from jax.experimental import pallas as pl
from jax.experimental.pallas import tpu as pltpu
from jax.experimental.pallas import tpu_sc as plsc

EMBED = 64
NUM_WORKERS = 32
CHUNK = 200
NBUF = 4


def _gather_call(n_batch, n_hist, idx_flat, table):
    n_total = n_batch * n_hist
    b_per_w = n_total // NUM_WORKERS
    n_chunks = b_per_w // CHUNK
    n_groups = n_chunks // NBUF
    mesh = plsc.VectorSubcoreMesh(core_axis_name="c", subcore_axis_name="s")

    @functools.partial(
        pl.kernel,
        mesh=mesh,
        out_type=jax.ShapeDtypeStruct((n_batch, n_hist, EMBED), jnp.float32),
        scratch_types=[
            pltpu.VMEM((b_per_w,), jnp.int32),
            pltpu.VMEM((NBUF, CHUNK, 2 * EMBED), jnp.float32),
        ] + [pltpu.SemaphoreType.DMA] * (2 * NBUF),
        compiler_params=pltpu.CompilerParams(use_tc_tiling_on_sc=False),
    )
    def grab(idx_hbm, tab_hbm, out_hbm, idx_v, rows_v, *sems):
        gsem = sems[:NBUF]
        ssem = sems[NBUF:]
        wid = lax.axis_index("s") * 2 + lax.axis_index("c")
        base = wid * b_per_w

        pltpu.sync_copy(idx_hbm.at[pl.ds(base, b_per_w)], idx_v)

        nb = CHUNK // 200
        b0 = wid * (b_per_w // 200)

        def start_gather(i, b):
            pltpu.async_copy(
                tab_hbm.at[idx_v.at[pl.ds(i * CHUNK, CHUNK)]],
                rows_v.at[b], gsem[b])

        def wait_gather(b):
            pltpu.make_async_copy(
                tab_hbm.at[idx_v.at[pl.ds(0, CHUNK)]], rows_v.at[b],
                gsem[b]).wait()

        def start_store(i, b):
            pltpu.async_copy(
                rows_v.at[b, :, pl.ds(0, EMBED)], out_hbm.at[b0 + i],
                ssem[b])

        def wait_store(b):
            pltpu.make_async_copy(
                rows_v.at[b, :, pl.ds(0, EMBED)], out_hbm.at[b0],
                ssem[b]).wait()

        for b in range(NBUF):
            start_gather(b, b)

        def body(j, carry):
            i0 = j * NBUF
            for b in range(NBUF):
                i = i0 + b
                wait_gather(b)
                start_store(i, b)
                @pl.when(i + NBUF < n_chunks)
                def _():
                    wait_store(b)
                    start_gather(i + NBUF, b)
            return carry

        lax.fori_loop(0, n_groups, body, 0)
        for b in range(NBUF):
            wait_store(b)

    return grab(idx_flat, table)


def kernel(input, table):
    b, h = input.shape
    idx_flat = input.reshape(b * h).astype(jnp.int32)
    tab_p = jnp.pad(table, ((0, 0), (0, EMBED)))
    return _gather_call(b, h, idx_flat, tab_p)


# speedup vs baseline: 1.3437x; 1.3437x over previous
import functools

import jax
import jax.numpy as jnp
from jax import lax
from jax.experimental import pallas as pl
from jax.experimental.pallas import tpu as pltpu
from jax.experimental.pallas import tpu_sc as plsc

EMBED = 64
NUM_WORKERS = 32
CHUNK = 400
NBUF = 4


def _gather_call(n_batch, n_hist, idx_flat, table):
    n_total = n_batch * n_hist
    b_per_w = n_total // NUM_WORKERS
    n_chunks = b_per_w // CHUNK
    n_groups = n_chunks // NBUF
    mesh = plsc.VectorSubcoreMesh(core_axis_name="c", subcore_axis_name="s")

    @functools.partial(
        pl.kernel,
        mesh=mesh,
        out_type=jax.ShapeDtypeStruct((n_batch, n_hist, 2 * EMBED), jnp.float32),
        scratch_types=[
            pltpu.VMEM((b_per_w,), jnp.int32),
            pltpu.VMEM((NBUF, CHUNK, EMBED), jnp.float32),
        ] + [pltpu.SemaphoreType.DMA] * (2 * NBUF),
        compiler_params=pltpu.CompilerParams(use_tc_tiling_on_sc=False),
    )
    def grab(idx_hbm, tab_hbm, out_hbm, idx_v, rows_v, *sems):
        gsem = sems[:NBUF]
        ssem = sems[NBUF:]
        wid = lax.axis_index("s") * 2 + lax.axis_index("c")
        base = wid * b_per_w

        pltpu.sync_copy(idx_hbm.at[pl.ds(base, b_per_w)], idx_v)

        nb = CHUNK // 200
        b0 = wid * (b_per_w // 200)

        def start_gather(i, b):
            pltpu.async_copy(
                tab_hbm.at[idx_v.at[pl.ds(i * CHUNK, CHUNK)]],
                rows_v.at[b], gsem[b])

        def wait_gather(b):
            pltpu.make_async_copy(
                tab_hbm.at[idx_v.at[pl.ds(0, CHUNK)]], rows_v.at[b],
                gsem[b]).wait()

        def start_store(i, b):
            for k in range(nb):
                pltpu.async_copy(
                    rows_v.at[b, pl.ds(k * n_hist, n_hist)],
                    out_hbm.at[b0 + i * nb + k, :, pl.ds(0, EMBED)], ssem[b])

        def wait_store(b):
            for k in range(nb):
                pltpu.make_async_copy(
                    rows_v.at[b, pl.ds(k * n_hist, n_hist)],
                    out_hbm.at[b0 + k, :, pl.ds(0, EMBED)], ssem[b]).wait()

        for b in range(NBUF):
            start_gather(b, b)

        def body(j, carry):
            i0 = j * NBUF
            for b in range(NBUF):
                i = i0 + b
                wait_gather(b)
                start_store(i, b)
                @pl.when(i + NBUF < n_chunks)
                def _():
                    wait_store(b)
                    start_gather(i + NBUF, b)
            return carry

        lax.fori_loop(0, n_groups, body, 0)
        for b in range(NBUF):
            wait_store(b)

    return grab(idx_flat, table)


def kernel(input, table):
    b, h = input.shape
    idx_flat = input.reshape(b * h).astype(jnp.int32)
    out_p = _gather_call(b, h, idx_flat, table)
    return out_p[:, :, :EMBED]
